# Initial kernel scaffold; baseline (speedup 1.0000x reference)
import jax
import jax.numpy as jnp
from jax.experimental import pallas as pl


def kernel(video_clips):
    raise NotImplementedError("write your pallas kernel here")


# speedup vs baseline: 5.9598x; 5.9598x over previous
import numpy as np
import jax
import jax.numpy as jnp
from jax import lax
from jax.experimental import pallas as pl
from jax.experimental.pallas import tpu as pltpu
from jax.experimental.pallas import tpu_sc as plsc

B, C, T, H, W = 32, 3, 16, 112, 112
HW = H * W
EPS = 1e-08
KSIZE = 11
PAD = KSIZE // 2
SIGMA = KSIZE / 3.0
NCHUNK = HW // 16
KFG = HW // 2
KBG = HW // 10
NBINS = 1008


def _gauss2d():
    ax = jnp.arange(KSIZE, dtype=jnp.float32) - KSIZE // 2
    g = jnp.exp(-(ax ** 2) / (2.0 * SIGMA ** 2))
    g = g / jnp.sum(g)
    return jnp.outer(g, g)


def _blur(x):
    k = _gauss2d()[None, None]
    xp = jnp.pad(x[:, None], ((0, 0), (0, 0), (PAD, PAD), (PAD, PAD)), mode='reflect')
    return jax.lax.conv_general_dilated(
        xp, k, (1, 1), 'VALID', dimension_numbers=('NCHW', 'OIHW', 'NCHW'))[:, 0]


def _vmean_body(v_ref, vmean_ref):
    v = v_ref[0]
    m = v[:, 0]
    for t in range(1, T):
        m = m + v[:, t]
    vmean_ref[0] = m * jnp.float32(1.0 / T)


def _vmean(video):
    return pl.pallas_call(
        _vmean_body,
        grid=(B,),
        in_specs=[pl.BlockSpec((1, C, T, H, W), lambda b: (b, 0, 0, 0, 0))],
        out_specs=pl.BlockSpec((1, C, H, W), lambda b: (b, 0, 0, 0)),
        out_shape=jax.ShapeDtypeStruct((B, C, H, W), jnp.float32),
    )(video)


def _cm1_body(vmean_ref, arg_ref, s_ref, v_ref):
    vm = vmean_ref[0]
    r, g, b = vm[0], vm[1], vm[2]
    maxc = jnp.maximum(jnp.maximum(r, g), b)
    minc = jnp.minimum(jnp.minimum(r, g), b)
    deltac = maxc - minc
    s = deltac / (maxc + EPS)
    ds = jnp.where(deltac == 0, jnp.ones_like(deltac), deltac)
    rc = (maxc - r) / ds
    gc = (maxc - g) / ds
    bc = (maxc - b) / ds
    h6 = jnp.where(r >= maxc, bc - gc,
                   jnp.where(g >= maxc, (rc - bc) + 2.0, (gc - rc) + 4.0))
    h = 2.0 * np.pi * ((h6 / 6.0) % 1.0)
    arg_ref[0] = (h * 2) * np.float32(np.pi)
    s_ref[0] = s
    v_ref[0] = maxc


def _cm2_body(s_ref, v_ref, cos_ref, sin_ref, cm_ref):
    s = s_ref[0]
    vv = v_ref[0]
    hx = (s * cos_ref[0] + 1) / 2
    hy = (s * sin_ref[0] + 1) / 2
    hq = jnp.round(hx * 9 + 1)
    sq = jnp.round(hy * 9 + 1)
    vq = jnp.round(vv * 9 + 1)
    cm_ref[0] = (hq + (sq - 1) * 10 + (vq - 1) * 100).astype(jnp.int32)


def _colormap(vmean):
    arg, s, vv = pl.pallas_call(
        _cm1_body,
        grid=(B,),
        in_specs=[pl.BlockSpec((1, C, H, W), lambda b: (b, 0, 0, 0))],
        out_specs=[pl.BlockSpec((1, H, W), lambda b: (b, 0, 0))] * 3,
        out_shape=[jax.ShapeDtypeStruct((B, H, W), jnp.float32)] * 3,
    )(vmean)
    cosv, sinv = jnp.cos(arg), jnp.sin(arg)
    return pl.pallas_call(
        _cm2_body,
        grid=(B,),
        in_specs=[pl.BlockSpec((1, H, W), lambda b: (b, 0, 0))] * 4,
        out_specs=pl.BlockSpec((1, H, W), lambda b: (b, 0, 0)),
        out_shape=jax.ShapeDtypeStruct((B, H, W), jnp.int32),
    )(s, vv, cosv, sinv).reshape(B, HW)


def _radix_kth(m_ref, hist_ref, k, largest):
    prefix = jnp.int32(0)
    krem = jnp.int32(k)
    ones = jnp.ones((16,), jnp.int32)
    zeros = jnp.zeros((16,), jnp.int32)
    for p in range(4):
        shift = 24 - 8 * p
        for j in range(256 // 16):
            hist_ref[pl.ds(j * 16, 16)] = zeros

        def body(i, _):
            bits = m_ref[pl.ds(i * 16, 16)]
            idx = lax.shift_right_logical(bits, jnp.int32(shift)) & jnp.int32(255)
            if p == 0:
                plsc.addupdate_scatter(hist_ref, [idx], ones)
            else:
                hi = lax.shift_right_logical(bits, jnp.int32(shift + 8))
                plsc.addupdate_scatter(hist_ref, [idx], ones, mask=hi == prefix)
            return 0

        lax.fori_loop(0, NCHUNK, body, 0, unroll=4)

        carry = jnp.int32(0)
        bstar = jnp.int32(0)
        cum_sel = jnp.int32(0)
        groups = range(15, -1, -1) if largest else range(16)
        for g in groups:
            vec = hist_ref[pl.ds(g * 16, 16)]
            if largest:
                cs = lax.rev(plsc.cumsum(lax.rev(vec, (0,))), (0,))
            else:
                cs = plsc.cumsum(vec)
            cum_incl = cs + carry
            cum_before = cum_incl - vec
            hit = jnp.logical_and(cum_before < krem, cum_incl >= krem)
            bin_ids = lax.iota(jnp.int32, 16) + jnp.int32(g * 16)
            bstar = bstar + jnp.sum(jnp.where(hit, bin_ids, zeros))
            cum_sel = cum_sel + jnp.sum(jnp.where(hit, cum_before, zeros))
            carry = carry + jnp.sum(vec)
        krem = krem - cum_sel
        prefix = (prefix << jnp.int32(8)) | bstar
    return prefix


def _sc_wid():
    info = plsc.get_sparse_core_info()
    return lax.axis_index("s") * info.num_cores + lax.axis_index("c")


def _refine_body(m_hbm, c_hbm, out_hbm, m_v, c_v, r_v, hist_v, hf_v, hb_v):
    wid = _sc_wid()
    pltpu.sync_copy(m_hbm.at[pl.ds(wid * HW, HW)], m_v)
    pltpu.sync_copy(c_hbm.at[pl.ds(wid * HW, HW)], c_v)

    t_fg = _radix_kth(m_v, hist_v, KFG, True)
    t_bg = _radix_kth(m_v, hist_v, KBG, False)

    zf = jnp.zeros((16,), jnp.float32)
    onesf = jnp.ones((16,), jnp.float32)
    for j in range(NBINS // 16):
        hf_v[pl.ds(j * 16, 16)] = zf
        hb_v[pl.ds(j * 16, 16)] = zf

    def hist_body(i, _):
        bits = m_v[pl.ds(i * 16, 16)]
        cv = c_v[pl.ds(i * 16, 16)]
        valid = cv < 1000
        sel_fg = jnp.logical_and(bits >= t_fg, valid)
        sel_bg = jnp.logical_and(bits <= t_bg, valid)
        plsc.addupdate_scatter(hf_v, [cv], onesf, mask=sel_fg)
        plsc.addupdate_scatter(hb_v, [cv], onesf, mask=sel_bg)
        return 0

    lax.fori_loop(0, NCHUNK, hist_body, 0, unroll=4)

    sf = jnp.zeros((16,), jnp.float32)
    sb = jnp.zeros((16,), jnp.float32)
    for j in range(NBINS // 16):
        sf = sf + hf_v[pl.ds(j * 16, 16)]
        sb = sb + hb_v[pl.ds(j * 16, 16)]
    sumf = jnp.sum(sf) + jnp.float32(EPS)
    sumb = (jnp.sum(sb) + jnp.float32(1000.0)) + jnp.float32(EPS)

    for j in range(NBINS // 16):
        hf_v[pl.ds(j * 16, 16)] = hf_v[pl.ds(j * 16, 16)] / sumf
        hb_v[pl.ds(j * 16, 16)] = (hb_v[pl.ds(j * 16, 16)] + 1.0) / sumb

    def pr_body(i, _):
        cv = c_v[pl.ds(i * 16, 16)]
        idx = jnp.minimum(cv, 999)
        pf = plsc.load_gather(hf_v, [idx])
        pb = plsc.load_gather(hb_v, [idx])
        r_v[pl.ds(i * 16, 16)] = pf / (pb + pf)
        return 0

    lax.fori_loop(0, NCHUNK, pr_body, 0, unroll=4)
    pltpu.sync_copy(r_v, out_hbm.at[pl.ds(wid * HW, HW)])


def _sc_refine(m1, color):
    mesh = plsc.VectorSubcoreMesh(core_axis_name="c", subcore_axis_name="s")
    f = pl.kernel(
        _refine_body,
        mesh=mesh,
        compiler_params=pltpu.CompilerParams(needs_layout_passes=False),
        out_type=jax.ShapeDtypeStruct((B * HW,), jnp.float32),
        scratch_types=[
            pltpu.VMEM((HW,), jnp.int32),
            pltpu.VMEM((HW,), jnp.int32),
            pltpu.VMEM((HW,), jnp.float32),
            pltpu.VMEM((256,), jnp.int32),
            pltpu.VMEM((NBINS,), jnp.float32),
            pltpu.VMEM((NBINS,), jnp.float32),
        ],
    )
    m1bits = lax.bitcast_convert_type(m1.reshape(-1), jnp.int32)
    return f(m1bits, color.reshape(-1)).reshape(B, HW)


def _mask_body(m_hbm, out_hbm, m_v, r_v, hist_v):
    wid = _sc_wid()
    pltpu.sync_copy(m_hbm.at[pl.ds(wid * HW, HW)], m_v)
    t = _radix_kth(m_v, hist_v, KFG, True)
    onesf = jnp.ones((16,), jnp.float32)
    zf = jnp.zeros((16,), jnp.float32)

    def body(i, _):
        bits = m_v[pl.ds(i * 16, 16)]
        r_v[pl.ds(i * 16, 16)] = jnp.where(bits >= t, onesf, zf)
        return 0

    lax.fori_loop(0, NCHUNK, body, 0, unroll=4)
    pltpu.sync_copy(r_v, out_hbm.at[pl.ds(wid * HW, HW)])


def _sc_mask(m2):
    mesh = plsc.VectorSubcoreMesh(core_axis_name="c", subcore_axis_name="s")
    f = pl.kernel(
        _mask_body,
        mesh=mesh,
        compiler_params=pltpu.CompilerParams(needs_layout_passes=False),
        out_type=jax.ShapeDtypeStruct((B * HW,), jnp.float32),
        scratch_types=[
            pltpu.VMEM((HW,), jnp.int32),
            pltpu.VMEM((HW,), jnp.float32),
            pltpu.VMEM((256,), jnp.int32),
        ],
    )
    m2bits = lax.bitcast_convert_type(m2.reshape(-1), jnp.int32)
    return f(m2bits).reshape(B, H, W)


def _fuse_body(perm_ref, v_ref, vp_ref, m_ref, o_ref):
    m = m_ref[0, 0]
    o_ref[0, :, 0] = vp_ref[0, :, 0] * (1.0 - m) + v_ref[0, :, 0] * m


def _fuse(video, mask, perm):
    grid_spec = pltpu.PrefetchScalarGridSpec(
        num_scalar_prefetch=1,
        grid=(B, T),
        in_specs=[
            pl.BlockSpec((1, C, 1, H, W), lambda b, t, perm: (b, 0, t, 0, 0)),
            pl.BlockSpec((1, C, 1, H, W), lambda b, t, perm: (perm[b], 0, t, 0, 0)),
            pl.BlockSpec((1, 1, H, W), lambda b, t, perm: (b, 0, 0, 0)),
        ],
        out_specs=pl.BlockSpec((1, C, 1, H, W), lambda b, t, perm: (b, 0, t, 0, 0)),
    )
    return pl.pallas_call(
        _fuse_body,
        grid_spec=grid_spec,
        out_shape=jax.ShapeDtypeStruct((B, C, T, H, W), jnp.float32),
    )(perm, video, video, mask.reshape(B, 1, H, W))


def kernel(video_clips):
    im_diff = jnp.mean(jnp.sum(jnp.abs(video_clips[:, :, :-1] - video_clips[:, :, 1:]), axis=1), axis=1)
    vmean = _vmean(video_clips)
    color = _colormap(vmean)
    mb1 = _blur(im_diff).reshape(B, HW)
    refine = _sc_refine(mb1, color)
    mb2 = _blur(refine.reshape(B, H, W)).reshape(B, HW)
    mask = _sc_mask(mb2)
    perm = jax.random.permutation(jax.random.key(42), B)
    return _fuse(video_clips, mask, perm)
